# Initial kernel scaffold; baseline (speedup 1.0000x reference)
import jax
import jax.numpy as jnp
from jax.experimental import pallas as pl


def kernel(x, edge_index, batch, W1, b1, W2, b2, W3, b3, Wfc1, bfc1, Wfc2, bfc2):
    raise NotImplementedError("write your pallas kernel here")


# speedup vs baseline: 10.3507x; 10.3507x over previous
import functools

import jax
import jax.numpy as jnp
from jax import lax
from jax.experimental import pallas as pl
from jax.experimental.pallas import tpu as pltpu
import jax.experimental.pallas.tpu_sc as plsc

N = 10000
E = 320000
NG = 64
NW = 32
K = 128
NB = 79
EPW = NB * K
NPAD = 10240
RPS = NPAD // 16
BN = 400
GRID = N // BN


def _deg_body(dst_hbm, ones_hbm, zeros_hbm, out_hbm, dst_v, ones_v, deg_sh, sem):
    cid = lax.axis_index("c")
    sid = lax.axis_index("s")
    wid = cid * 16 + sid
    pltpu.sync_copy(dst_hbm.at[wid], dst_v)
    pltpu.sync_copy(ones_hbm, ones_v)
    pltpu.sync_copy(zeros_hbm, deg_sh.at[pl.ds(sid * RPS, RPS)])
    plsc.subcore_barrier()

    def step(j, carry):
        pltpu.sync_copy(ones_v, deg_sh.at[dst_v.at[j]], add=True)
        return carry

    lax.fori_loop(0, NB, step, 0)
    plsc.subcore_barrier()
    pltpu.sync_copy(deg_sh.at[pl.ds(sid * RPS, RPS)],
                    out_hbm.at[cid, pl.ds(sid * RPS, RPS)])


@functools.lru_cache(maxsize=None)
def _build_sc_kernels():
    mesh = plsc.VectorSubcoreMesh(core_axis_name="c", subcore_axis_name="s")
    deg = pl.kernel(
        _deg_body,
        out_type=jax.ShapeDtypeStruct((2, NPAD, 128), jnp.float32),
        mesh=mesh,
        scratch_types=[
            pltpu.VMEM((NB, K), jnp.int32),
            pltpu.VMEM((K, 128), jnp.float32),
            pltpu.VMEM_SHARED((NPAD, 128), jnp.float32),
            pltpu.SemaphoreType.DMA,
        ],
    )
    prop = pl.kernel(
        _prop_body,
        out_type=jax.ShapeDtypeStruct((2, NPAD, 128), jnp.float32),
        mesh=mesh,
        scratch_types=[
            pltpu.VMEM((NB, K), jnp.int32),
            pltpu.VMEM((NB, K), jnp.int32),
            pltpu.VMEM((K, 128), jnp.float32),
            pltpu.VMEM_SHARED((NPAD, 128), jnp.float32),
            pltpu.SemaphoreType.DMA,
        ],
    )
    return deg, prop


def _sc_degree(dst, ones16, z16):
    return _build_sc_kernels()[0](dst, ones16, z16)


def _sc_prop(y, src, dst, z128):
    return _build_sc_kernels()[1](y, src, dst, z128)


def _prop_body(y_hbm, src_hbm, dst_hbm, zeros_hbm, out_hbm,
               src_v, dst_v, rows_v, agg_sh, sem):
    cid = lax.axis_index("c")
    sid = lax.axis_index("s")
    wid = cid * 16 + sid
    pltpu.sync_copy(src_hbm.at[wid], src_v)
    pltpu.sync_copy(dst_hbm.at[wid], dst_v)
    pltpu.sync_copy(zeros_hbm, agg_sh.at[pl.ds(sid * RPS, RPS)])
    plsc.subcore_barrier()

    def step(j, carry):
        pltpu.async_copy(y_hbm.at[src_v.at[j]], rows_v, sem).wait()
        pltpu.sync_copy(rows_v, agg_sh.at[dst_v.at[j]], add=True)
        return carry

    lax.fori_loop(0, NB, step, 0)
    plsc.subcore_barrier()
    pltpu.sync_copy(agg_sh.at[pl.ds(sid * RPS, RPS)],
                    out_hbm.at[cid, pl.ds(sid * RPS, RPS)])


def _tc1_body(deg_ref, x_ref, w1_ref, y1_ref, dinv_ref):
    deg = deg_ref[0, :, 0:1] + deg_ref[1, :, 0:1] + 1.0
    dinv = lax.rsqrt(deg)
    xw = jnp.dot(x_ref[...], w1_ref[...], preferred_element_type=jnp.float32)
    y1_ref[...] = xw * dinv
    dinv_ref[...] = jnp.broadcast_to(dinv, xw.shape)


def _tc1(deg_parts, x, w1):
    return pl.pallas_call(
        _tc1_body,
        grid=(GRID,),
        in_specs=[
            pl.BlockSpec((2, BN, 128), lambda i: (0, i, 0)),
            pl.BlockSpec((BN, 128), lambda i: (i, 0)),
            pl.BlockSpec((128, 128), lambda i: (0, 0)),
        ],
        out_specs=[
            pl.BlockSpec((BN, 128), lambda i: (i, 0)),
            pl.BlockSpec((BN, 128), lambda i: (i, 0)),
        ],
        out_shape=[
            jax.ShapeDtypeStruct((N, 128), jnp.float32),
            jax.ShapeDtypeStruct((N, 128), jnp.float32),
        ],
    )(deg_parts, x, w1)


def _tc2_body(agg_ref, y1_ref, dinv_ref, b1_ref, y2_ref):
    agg = agg_ref[0] + agg_ref[1]
    dinv = dinv_ref[...]
    h1 = jnp.maximum(dinv * (agg + y1_ref[...]) + b1_ref[...], 0.0)
    y2_ref[...] = dinv * h1


def _tc2(agg_parts, y1, dinv, b1):
    return pl.pallas_call(
        _tc2_body,
        grid=(GRID,),
        in_specs=[
            pl.BlockSpec((2, BN, 128), lambda i: (0, i, 0)),
            pl.BlockSpec((BN, 128), lambda i: (i, 0)),
            pl.BlockSpec((BN, 128), lambda i: (i, 0)),
            pl.BlockSpec((1, 128), lambda i: (0, 0)),
        ],
        out_specs=pl.BlockSpec((BN, 128), lambda i: (i, 0)),
        out_shape=jax.ShapeDtypeStruct((N, 128), jnp.float32),
    )(agg_parts, y1, dinv, b1)


def _tc3_body(agg_ref, y2_ref, dinv_ref, w2_ref, b2_ref, w3_ref, y3_ref):
    agg = agg_ref[0] + agg_ref[1]
    dinv = dinv_ref[...]
    p2 = dinv * (agg + y2_ref[...])
    h2 = jnp.maximum(
        jnp.dot(p2, w2_ref[...], preferred_element_type=jnp.float32)
        + b2_ref[...], 0.0)
    xw3 = jnp.dot(h2, w3_ref[...], preferred_element_type=jnp.float32)
    y3 = jnp.concatenate([dinv, dinv], axis=1) * xw3
    y3_ref[0] = y3[:, :128]
    y3_ref[1] = y3[:, 128:]


def _tc3(agg_parts, y2, dinv, w2, b2, w3):
    return pl.pallas_call(
        _tc3_body,
        grid=(GRID,),
        in_specs=[
            pl.BlockSpec((2, BN, 128), lambda i: (0, i, 0)),
            pl.BlockSpec((BN, 128), lambda i: (i, 0)),
            pl.BlockSpec((BN, 128), lambda i: (i, 0)),
            pl.BlockSpec((128, 256), lambda i: (0, 0)),
            pl.BlockSpec((1, 256), lambda i: (0, 0)),
            pl.BlockSpec((256, 256), lambda i: (0, 0)),
        ],
        out_specs=pl.BlockSpec((2, BN, 128), lambda i: (0, i, 0)),
        out_shape=jax.ShapeDtypeStruct((2, N, 128), jnp.float32),
    )(agg_parts, y2, dinv, w2, b2, w3)


def _tc4_body(a0_ref, a1_ref, y3_ref, dinv_ref, b3_ref, batch_ref,
              wf1_ref, bf1_ref, wf2_ref, bf2_ref, out_ref,
              pooled_acc, count_acc):
    i = pl.program_id(0)

    @pl.when(i == 0)
    def _init():
        pooled_acc[...] = jnp.zeros_like(pooled_acc)
        count_acc[...] = jnp.zeros_like(count_acc)

    dinv = dinv_ref[...]
    h3a = dinv * (a0_ref[0] + a0_ref[1] + y3_ref[0])
    h3b = dinv * (a1_ref[0] + a1_ref[1] + y3_ref[1])
    h3 = jnp.maximum(jnp.concatenate([h3a, h3b], axis=1) + b3_ref[...], 0.0)

    b = batch_ref[0, 0]
    gids = lax.broadcasted_iota(jnp.int32, (NG, BN), 0)
    onehot = jnp.where(b[None, :] == gids, 1.0, 0.0)
    pooled_acc[...] += jnp.dot(onehot, h3, preferred_element_type=jnp.float32)
    count_acc[...] += jnp.broadcast_to(
        jnp.sum(onehot, axis=1, keepdims=True), count_acc.shape)

    @pl.when(i == pl.num_programs(0) - 1)
    def _final():
        cnt = jnp.maximum(count_acc[:, 0:1], 1.0)
        pooled = pooled_acc[...] / cnt
        g = jnp.maximum(
            jnp.dot(pooled, wf1_ref[...], preferred_element_type=jnp.float32)
            + bf1_ref[...], 0.0)
        out_ref[...] = (
            jnp.dot(g, wf2_ref[...], preferred_element_type=jnp.float32)
            + bf2_ref[...])


def _tc4(a0, a1, y3p, dinv, b3, batch3, wf1, bf1, wf2, bf2):
    return pl.pallas_call(
        _tc4_body,
        grid=(GRID,),
        in_specs=[
            pl.BlockSpec((2, BN, 128), lambda i: (0, i, 0)),
            pl.BlockSpec((2, BN, 128), lambda i: (0, i, 0)),
            pl.BlockSpec((2, BN, 128), lambda i: (0, i, 0)),
            pl.BlockSpec((BN, 128), lambda i: (i, 0)),
            pl.BlockSpec((1, 256), lambda i: (0, 0)),
            pl.BlockSpec((1, 1, BN), lambda i: (i, 0, 0)),
            pl.BlockSpec((256, 512), lambda i: (0, 0)),
            pl.BlockSpec((1, 512), lambda i: (0, 0)),
            pl.BlockSpec((512, 10), lambda i: (0, 0)),
            pl.BlockSpec((1, 10), lambda i: (0, 0)),
        ],
        out_specs=pl.BlockSpec((NG, 10), lambda i: (0, 0)),
        out_shape=jax.ShapeDtypeStruct((NG, 10), jnp.float32),
        scratch_shapes=[
            pltpu.VMEM((NG, 256), jnp.float32),
            pltpu.VMEM((NG, 128), jnp.float32),
        ],
    )(a0, a1, y3p, dinv, b3, batch3, wf1, bf1, wf2, bf2)


def _prep_edges(edge_index):
    src = edge_index[0].astype(jnp.int32).reshape(NW, E // NW)
    dst = edge_index[1].astype(jnp.int32).reshape(NW, E // NW)
    pad = EPW - E // NW
    src = jnp.pad(src, ((0, 0), (0, pad)), constant_values=0)
    dst = jnp.pad(dst, ((0, 0), (0, pad)), constant_values=NPAD - 1)
    return src.reshape(NW, NB, K), dst.reshape(NW, NB, K)


def kernel(x, edge_index, batch, W1, b1, W2, b2, W3, b3,
           Wfc1, bfc1, Wfc2, bfc2):
    src, dst = _prep_edges(edge_index)
    ones128 = jnp.ones((K, 128), jnp.float32)
    z128 = jnp.zeros((RPS, 128), jnp.float32)
    batch3 = batch.astype(jnp.int32).reshape(GRID, 1, BN)

    deg_parts = _sc_degree(dst, ones128, z128)
    y1, dinv = _tc1(deg_parts, x, W1)
    agg1 = _sc_prop(y1, src, dst, z128)
    y2 = _tc2(agg1, y1, dinv, b1.reshape(1, 128))
    agg2 = _sc_prop(y2, src, dst, z128)
    y3p = _tc3(agg2, y2, dinv, W2, b2.reshape(1, 256), W3)
    a30 = _sc_prop(y3p[0], src, dst, z128)
    a31 = _sc_prop(y3p[1], src, dst, z128)
    return _tc4(a30, a31, y3p, dinv, b3.reshape(1, 256), batch3,
                Wfc1, bfc1.reshape(1, 512), Wfc2, bfc2.reshape(1, 10))
